# baseline (speedup 1.0000x reference)
import jax
import jax.numpy as jnp
from jax import lax
from jax.experimental import pallas as pl
from jax.experimental.pallas import tpu as pltpu
from jax.experimental.pallas import tpu_sc as plsc
import functools

B = 4096
L = 50
LP = 56
D = 32
NC, NS = 2, 16
NW = NC * NS
BPW = B // NW


def _sc_pool(xp, w_word, w_bi, w_tri):
    mesh = plsc.VectorSubcoreMesh(core_axis_name="c", subcore_axis_name="s")

    @functools.partial(
        pl.kernel,
        out_type=jax.ShapeDtypeStruct((NW, BPW, 3 * D), jnp.float32),
        mesh=mesh,
        scratch_types=[
            pltpu.VMEM((BPW, LP), jnp.int32),
            pltpu.VMEM((LP, D), jnp.float32),
            pltpu.VMEM((LP, D), jnp.float32),
            pltpu.VMEM((LP, D), jnp.float32),
            pltpu.VMEM((BPW, 3 * D), jnp.float32),
            pltpu.SemaphoreType.DMA,
            pltpu.SemaphoreType.DMA,
            pltpu.SemaphoreType.DMA,
        ],
        compiler_params=pltpu.CompilerParams(use_tc_tiling_on_sc=False),
    )
    def k(x_hbm, ww_hbm, wb_hbm, wt_hbm, out_hbm,
          idx_v, g0, g1, g2, out_v, s0, s1, s2):
        wid = lax.axis_index("s") * NC + lax.axis_index("c")
        pltpu.sync_copy(x_hbm.at[wid], idx_v)

        def b_body(b, carry):
            c0 = pltpu.async_copy(ww_hbm.at[idx_v.at[b]], g0, s0)
            c1 = pltpu.async_copy(wb_hbm.at[idx_v.at[b]], g1, s1)
            c2 = pltpu.async_copy(wt_hbm.at[idx_v.at[b]], g2, s2)
            c0.wait()
            c1.wait()
            c2.wait()

            def l_body(l, acc):
                a0, a1, a2, a3, a4, a5 = acc
                return (a0 + g0[l, pl.ds(0, 16)],
                        a1 + g0[l, pl.ds(16, 16)],
                        a2 + g1[l, pl.ds(0, 16)],
                        a3 + g1[l, pl.ds(16, 16)],
                        a4 + g2[l, pl.ds(0, 16)],
                        a5 + g2[l, pl.ds(16, 16)])

            z = jnp.zeros((16,), jnp.float32)
            a0, a1, a2, a3, a4, a5 = lax.fori_loop(
                0, L, l_body, (z, z, z, z, z, z))
            out_v[b, pl.ds(0, 16)] = a0
            out_v[b, pl.ds(16, 16)] = a1
            out_v[b, pl.ds(32, 16)] = a2
            out_v[b, pl.ds(48, 16)] = a3
            out_v[b, pl.ds(64, 16)] = a4
            out_v[b, pl.ds(80, 16)] = a5
            return carry

        lax.fori_loop(0, BPW, b_body, 0)
        pltpu.sync_copy(out_v, out_hbm.at[wid])

    return k(xp, w_word, w_bi, w_tri)


def _mlp_body(s_ref, w1_ref, b1_ref, w2_ref, b2_ref, o_ref):
    wf = jnp.dot(w1_ref[...], w2_ref[...],
                 preferred_element_type=jnp.float32) * (1.0 / L)
    bias = jnp.dot(b1_ref[...], w2_ref[...],
                   preferred_element_type=jnp.float32) + b2_ref[...]
    y = jnp.dot(s_ref[...], wf, preferred_element_type=jnp.float32) + bias
    o_ref[...] = jnp.maximum(y, 0.0)


def _mlp_tc(sums, w1, b1, w2, b2):
    return pl.pallas_call(
        _mlp_body,
        out_shape=jax.ShapeDtypeStruct((B, 32), jnp.float32),
    )(sums, w1, b1.reshape(1, -1), w2, b2.reshape(1, -1))


@jax.jit
def kernel(x, W_word, W_bi, W_tri, W1, b1, W2, b2):
    xp = jnp.pad(x, ((0, 0), (0, LP - L)))
    xp = xp.reshape(NW, BPW, LP)
    sums = _sc_pool(xp, W_word, W_bi, W_tri)
    sums = sums.reshape(B, 3 * D)
    return _mlp_tc(sums, W1, b1, W2, b2)
